# Initial kernel scaffold; baseline (speedup 1.0000x reference)
import jax
import jax.numpy as jnp
from jax.experimental import pallas as pl


def kernel(xs, edge_index, W1_rel, b1_rel, W1_root, bn1_w, bn1_b, W2_rel, b2_rel, W2_root, bn2_w, bn2_b):
    raise NotImplementedError("write your pallas kernel here")


# speedup vs baseline: 6.7479x; 6.7479x over previous
import functools

import jax
import jax.numpy as jnp
from jax import lax
from jax.experimental import pallas as pl
from jax.experimental.pallas import tpu as pltpu
from jax.experimental.pallas import tpu_sc as plsc

N = 10000
E = 320000
D = 128
H = 128
O = 32

NC = 2
NS = 16
L = 16

CHUNK = 128
NCHUNKS = E // CHUNK
ROWS_PER_TILE = 624
TAIL_ROWS = N - NS * ROWS_PER_TILE


def _make_agg(width):
  mesh = plsc.VectorSubcoreMesh(core_axis_name="c", subcore_axis_name="s")

  @functools.partial(
      pl.kernel,
      mesh=mesh,
      compiler_params=pltpu.CompilerParams(use_tc_tiling_on_sc=False),
      out_type=jax.ShapeDtypeStruct((NC, N, width), jnp.float32),
      scratch_types=[
          pltpu.VMEM((CHUNK,), jnp.int32),
          pltpu.VMEM((CHUNK,), jnp.int32),
          pltpu.VMEM((CHUNK, width), jnp.float32),
          pltpu.VMEM_SHARED((N, width), jnp.float32),
          pltpu.SemaphoreType.DMA,
      ],
  )
  def agg(x_hbm, src_hbm, dst_hbm, out_hbm, src_v, dst_v, rows_v, acc, sem):
    cid = lax.axis_index("c")
    sid = lax.axis_index("s")

    def zrow(i, _):
      def zcol(j, _):
        rows_v[i, pl.ds(j * L, L)] = jnp.zeros((L,), jnp.float32)
        return 0
      return lax.fori_loop(0, width // L, zcol, 0)
    lax.fori_loop(0, CHUNK, zrow, 0)

    base = sid * ROWS_PER_TILE
    nfull = ROWS_PER_TILE // CHUNK
    rem = ROWS_PER_TILE - nfull * CHUNK
    for k in range(nfull):
      pltpu.sync_copy(rows_v, acc.at[pl.ds(base + k * CHUNK, CHUNK)])
    if rem:
      pltpu.sync_copy(rows_v.at[pl.ds(0, rem)],
                      acc.at[pl.ds(base + nfull * CHUNK, rem)])

    @pl.when(sid == NS - 1)
    def _():
      pltpu.sync_copy(rows_v.at[pl.ds(0, TAIL_ROWS)],
                      acc.at[pl.ds(NS * ROWS_PER_TILE, TAIL_ROWS)])
    plsc.subcore_barrier()

    per_core = NCHUNKS // NC
    lo = cid * per_core + (sid * per_core) // NS
    hi = cid * per_core + ((sid + 1) * per_core) // NS

    def body(k, _):
      off = k * CHUNK
      pltpu.sync_copy(src_hbm.at[pl.ds(off, CHUNK)], src_v)
      pltpu.sync_copy(dst_hbm.at[pl.ds(off, CHUNK)], dst_v)
      pltpu.async_copy(x_hbm.at[src_v], rows_v, sem).wait()
      pltpu.sync_copy(rows_v, acc.at[dst_v], add=True)
      return 0
    lax.fori_loop(lo, hi, body, 0)
    plsc.subcore_barrier()

    pltpu.sync_copy(acc.at[pl.ds(base, ROWS_PER_TILE)],
                    out_hbm.at[cid, pl.ds(base, ROWS_PER_TILE)])

    @pl.when(sid == NS - 1)
    def _():
      pltpu.sync_copy(acc.at[pl.ds(NS * ROWS_PER_TILE, TAIL_ROWS)],
                      out_hbm.at[cid, pl.ds(NS * ROWS_PER_TILE, TAIL_ROWS)])

  return agg


_agg_d = _make_agg(D)
_agg_o = _make_agg(O)


def _tc1_body(agg_ref, xs_ref, w1r_ref, b1_ref, w1s_ref, bnw_ref, bnb_ref,
              w2r_ref, w2s_ref, z2_ref, z2root_ref):
  agg = agg_ref[0] + agg_ref[1]
  h = lax.dot_general(agg, w1r_ref[...], (((1,), (1,)), ((), ())),
                      preferred_element_type=jnp.float32)
  h += lax.dot_general(xs_ref[...], w1s_ref[...], (((1,), (1,)), ((), ())),
                       preferred_element_type=jnp.float32)
  h += b1_ref[...]
  m = jnp.mean(h, axis=0, keepdims=True)
  c = h - m
  v = jnp.mean(c * c, axis=0, keepdims=True)
  x1 = jnp.maximum(c * lax.rsqrt(v + 1e-5) * bnw_ref[...] + bnb_ref[...], 0.0)
  z2_ref[...] = lax.dot_general(x1, w2r_ref[...], (((1,), (1,)), ((), ())),
                                preferred_element_type=jnp.float32)
  z2root_ref[...] = lax.dot_general(x1, w2s_ref[...], (((1,), (1,)), ((), ())),
                                    preferred_element_type=jnp.float32)


_tc1 = pl.pallas_call(
    _tc1_body,
    out_shape=[jax.ShapeDtypeStruct((N, O), jnp.float32),
               jax.ShapeDtypeStruct((N, O), jnp.float32)],
)


def _tc2_body(agg_ref, z2root_ref, b2_ref, bnw_ref, bnb_ref, out_ref):
  h = agg_ref[0] + agg_ref[1] + z2root_ref[...] + b2_ref[...]
  m = jnp.mean(h, axis=0, keepdims=True)
  c = h - m
  v = jnp.mean(c * c, axis=0, keepdims=True)
  out_ref[...] = jnp.maximum(
      c * lax.rsqrt(v + 1e-5) * bnw_ref[...] + bnb_ref[...], 0.0)


_tc2 = pl.pallas_call(
    _tc2_body,
    out_shape=jax.ShapeDtypeStruct((N, O), jnp.float32),
)


def kernel(xs, edge_index, W1_rel, b1_rel, W1_root, bn1_w, bn1_b,
           W2_rel, b2_rel, W2_root, bn2_w, bn2_b):
  src = edge_index[0]
  dst = edge_index[1]
  agg1 = _agg_d(xs, src, dst)
  z2, z2root = _tc1(agg1, xs, W1_rel, b1_rel.reshape(1, H), W1_root,
                    bn1_w.reshape(1, H), bn1_b.reshape(1, H), W2_rel, W2_root)
  agg2 = _agg_o(z2, src, dst)
  return _tc2(agg2, z2root, b2_rel.reshape(1, O),
              bn2_w.reshape(1, O), bn2_b.reshape(1, O))
